# baseline (speedup 1.0000x reference)
import functools

import jax
import jax.numpy as jnp
from jax import lax
from jax.experimental import pallas as pl
from jax.experimental.pallas import tpu as pltpu
from jax.experimental.pallas import tpu_sc as plsc

_B, _C, _H, _W = 8, 128, 224, 224
_HW = _H * _W
_V = 1000
_VP = 1024
_NC, _NS = 2, 16
_NW = _NC * _NS
_CBLK = 16
_NCB = _C // _CBLK
_BPW = _B * _NCB // _NW
_CH = 1792
_NCHUNK = _HW // _CH
_T = _BPW * _NCHUNK


def _transpose_table(tpad):
    def body(t_ref, o_ref):
        o_ref[...] = t_ref[...].T

    return pl.pallas_call(
        body, out_shape=jax.ShapeDtypeStruct((_C, _VP), jnp.float32)
    )(tpad)


def _sc_gather(table_t, idx):
    mesh = plsc.VectorSubcoreMesh(
        core_axis_name="c", subcore_axis_name="s",
        num_cores=_NC, num_subcores=_NS)

    @functools.partial(
        pl.kernel,
        out_type=jax.ShapeDtypeStruct((_B * _C * _HW,), jnp.float32),
        mesh=mesh,
        compiler_params=pltpu.CompilerParams(needs_layout_passes=False),
        scratch_types=[
            pltpu.VMEM((_CBLK * _VP,), jnp.float32),
            pltpu.VMEM((_CH,), jnp.int32),
            pltpu.VMEM((_CH,), jnp.int32),
            pltpu.VMEM((_CBLK * _CH,), jnp.float32),
            pltpu.VMEM((_CBLK * _CH,), jnp.float32),
            pltpu.SemaphoreType.DMA,
            pltpu.SemaphoreType.DMA,
            pltpu.SemaphoreType.DMA,
            pltpu.SemaphoreType.DMA,
            pltpu.SemaphoreType.DMA,
        ],
    )
    def k(tt_hbm, idx_hbm, out_hbm, lut_v, idx0, idx1, st0, st1,
          sem_lut, sem_i0, sem_i1, sem_o0, sem_o1):
        wid = lax.axis_index("s") * _NC + lax.axis_index("c")
        cblk = wid // (_NW // _NCB)
        bpair = wid % (_NW // _NCB)
        idx_bufs = (idx0, idx1)
        stages = (st0, st1)
        sem_is = (sem_i0, sem_i1)
        sem_os = (sem_o0, sem_o1)

        def idx_off(t):
            b = bpair * _BPW + t // _NCHUNK
            return b * _HW + (t % _NCHUNK) * _CH

        def out_off(t):
            b = bpair * _BPW + t // _NCHUNK
            return b * (_C * _HW) + (cblk * _CBLK) * _HW + (t % _NCHUNK) * _CH

        lut_copy = pltpu.async_copy(
            tt_hbm.at[pl.ds(cblk * (_CBLK * _VP), _CBLK * _VP)], lut_v,
            sem_lut)
        pltpu.async_copy(idx_hbm.at[pl.ds(idx_off(0), _CH)], idx0, sem_i0)
        pltpu.async_copy(idx_hbm.at[pl.ds(idx_off(1), _CH)], idx1, sem_i1)
        lut_copy.wait()

        def chunk_body(t, _):
            for s in (0, 1):
                te = t + s
                idx_v, stage_v = idx_bufs[s], stages[s]
                pltpu.make_async_copy(
                    idx_hbm.at[pl.ds(0, _CH)], idx_v, sem_is[s]).wait()
                @pl.when(te >= 2)
                def _drain_out():
                    pltpu.make_async_copy(
                        stage_v, out_hbm.at[pl.ds(0, _CBLK * _CH)],
                        sem_os[s]).wait()

                @plsc.parallel_loop(0, _CH // 16, unroll=8)
                def _pos_body(i):
                    iv = idx_v[pl.ds(i * 16, 16)]
                    for c in range(_CBLK):
                        stage_v[pl.ds(c * _CH + i * 16, 16)] = (
                            plsc.load_gather(lut_v, [iv + c * _VP]))
                obase = out_off(te)
                for c in range(_CBLK):
                    pltpu.async_copy(
                        stage_v.at[pl.ds(c * _CH, _CH)],
                        out_hbm.at[pl.ds(obase + c * _HW, _CH)], sem_os[s])
                @pl.when(te + 2 < _T)
                def _prefetch():
                    pltpu.async_copy(
                        idx_hbm.at[pl.ds(idx_off(te + 2), _CH)], idx_v,
                        sem_is[s])
            return 0

        lax.fori_loop(0, _T // 2, lambda u, c: chunk_body(u * 2, c), 0)
        for s in (0, 1):
            pltpu.make_async_copy(
                stages[s], out_hbm.at[pl.ds(0, _CBLK * _CH)], sem_os[s]).wait()

    return k(table_t, idx)


def kernel(x, table):
    idx = x.reshape(_B * _HW)
    tpad = jnp.zeros((_VP, _C), jnp.float32).at[:_V].set(table)
    table_t = _transpose_table(tpad).reshape(_C * _VP)
    out = _sc_gather(table_t, idx)
    return out.reshape(_B, _C, _H, _W)


# speedup vs baseline: 1.0136x; 1.0136x over previous
import functools

import jax
import jax.numpy as jnp
from jax import lax
from jax.experimental import pallas as pl
from jax.experimental.pallas import tpu as pltpu
from jax.experimental.pallas import tpu_sc as plsc

_B, _C, _H, _W = 8, 128, 224, 224
_HW = _H * _W
_V = 1000
_VP = 1024
_NC, _NS = 2, 16
_NW = _NC * _NS
_CBLK = 16
_NCB = _C // _CBLK
_BPW = _B * _NCB // _NW
_CH = 1792
_NCHUNK = _HW // _CH
_T = _BPW * _NCHUNK


def _transpose_table(tpad):
    def body(t_ref, o_ref):
        o_ref[...] = t_ref[...].T

    return pl.pallas_call(
        body, out_shape=jax.ShapeDtypeStruct((_C, _VP), jnp.float32)
    )(tpad)


def _sc_gather(table_t, idx):
    mesh = plsc.VectorSubcoreMesh(
        core_axis_name="c", subcore_axis_name="s",
        num_cores=_NC, num_subcores=_NS)

    @functools.partial(
        pl.kernel,
        out_type=jax.ShapeDtypeStruct((_B * _C * _HW,), jnp.float32),
        mesh=mesh,
        compiler_params=pltpu.CompilerParams(needs_layout_passes=False),
        scratch_types=[
            pltpu.VMEM((_CBLK * _VP,), jnp.float32),
            pltpu.VMEM((_CH,), jnp.int32),
            pltpu.VMEM((_CH,), jnp.int32),
            pltpu.VMEM((_CBLK * _CH,), jnp.float32),
            pltpu.VMEM((_CBLK * _CH,), jnp.float32),
            pltpu.SemaphoreType.DMA,
            pltpu.SemaphoreType.DMA,
            pltpu.SemaphoreType.DMA,
            pltpu.SemaphoreType.DMA,
            pltpu.SemaphoreType.DMA,
        ],
    )
    def k(tt_hbm, idx_hbm, out_hbm, lut_v, idx0, idx1, st0, st1,
          sem_lut, sem_i0, sem_i1, sem_o0, sem_o1):
        wid = lax.axis_index("s") * _NC + lax.axis_index("c")
        cblk = wid // (_NW // _NCB)
        bpair = wid % (_NW // _NCB)
        idx_bufs = (idx0, idx1)
        stages = (st0, st1)
        sem_is = (sem_i0, sem_i1)
        sem_os = (sem_o0, sem_o1)

        def idx_off(t):
            b = bpair * _BPW + t // _NCHUNK
            return b * _HW + (t % _NCHUNK) * _CH

        def out_off(t):
            b = bpair * _BPW + t // _NCHUNK
            return b * (_C * _HW) + (cblk * _CBLK) * _HW + (t % _NCHUNK) * _CH

        lut_copy = pltpu.async_copy(
            tt_hbm.at[pl.ds(cblk * (_CBLK * _VP), _CBLK * _VP)], lut_v,
            sem_lut)
        pltpu.async_copy(idx_hbm.at[pl.ds(idx_off(0), _CH)], idx0, sem_i0)
        pltpu.async_copy(idx_hbm.at[pl.ds(idx_off(1), _CH)], idx1, sem_i1)
        lut_copy.wait()

        def chunk_body(t, _):
            for s in (0, 1):
                te = t + s
                idx_v, stage_v = idx_bufs[s], stages[s]
                pltpu.make_async_copy(
                    idx_hbm.at[pl.ds(0, _CH)], idx_v, sem_is[s]).wait()
                @pl.when(te >= 2)
                def _drain_out():
                    pltpu.make_async_copy(
                        stage_v, out_hbm.at[pl.ds(0, _CBLK * _CH)],
                        sem_os[s]).wait()

                @plsc.parallel_loop(0, _CH // 16, unroll=4)
                def _pos_body(i):
                    iv = idx_v[pl.ds(i * 16, 16)]
                    for c in range(_CBLK):
                        stage_v[pl.ds(c * _CH + i * 16, 16)] = (
                            plsc.load_gather(lut_v, [iv + c * _VP]))
                obase = out_off(te)
                for c in range(_CBLK):
                    pltpu.async_copy(
                        stage_v.at[pl.ds(c * _CH, _CH)],
                        out_hbm.at[pl.ds(obase + c * _HW, _CH)], sem_os[s])
                @pl.when(te + 2 < _T)
                def _prefetch():
                    pltpu.async_copy(
                        idx_hbm.at[pl.ds(idx_off(te + 2), _CH)], idx_v,
                        sem_is[s])
            return 0

        lax.fori_loop(0, _T // 2, lambda u, c: chunk_body(u * 2, c), 0)
        for s in (0, 1):
            pltpu.make_async_copy(
                stages[s], out_hbm.at[pl.ds(0, _CBLK * _CH)], sem_os[s]).wait()

    return k(table_t, idx)


def kernel(x, table):
    idx = x.reshape(_B * _HW)
    tpad = jnp.zeros((_VP, _C), jnp.float32).at[:_V].set(table)
    table_t = _transpose_table(tpad).reshape(_C * _VP)
    out = _sc_gather(table_t, idx)
    return out.reshape(_B, _C, _H, _W)


# speedup vs baseline: 1.5808x; 1.5596x over previous
import functools

import jax
import jax.numpy as jnp
from jax import lax
from jax.experimental import pallas as pl
from jax.experimental.pallas import tpu as pltpu
from jax.experimental.pallas import tpu_sc as plsc

_B, _C, _H, _W = 8, 128, 224, 224
_HW = _H * _W
_V = 1000
_VP = 1024
_NC, _NS = 2, 16
_NW = _NC * _NS
_CBLK = 16
_NCB = _C // _CBLK
_BPW = _B * _NCB // _NW
_HB = 8
_CH = _HB * _W
_NCHUNK = _H // _HB
_T = _BPW * _NCHUNK
_WG = _W // 16


def _transpose_table(tpad):
    def body(t_ref, o_ref):
        o_ref[...] = t_ref[...].T

    return pl.pallas_call(
        body, out_shape=jax.ShapeDtypeStruct((_C, _VP), jnp.float32)
    )(tpad)


def _sc_gather(table_t, idx):
    mesh = plsc.VectorSubcoreMesh(
        core_axis_name="c", subcore_axis_name="s",
        num_cores=_NC, num_subcores=_NS)

    @functools.partial(
        pl.kernel,
        out_type=jax.ShapeDtypeStruct((_B, _C, _H, _W), jnp.float32),
        mesh=mesh,
        compiler_params=pltpu.CompilerParams(needs_layout_passes=False),
        scratch_types=[
            pltpu.VMEM((_CBLK * _VP,), jnp.float32),
            pltpu.VMEM((_CH,), jnp.int32),
            pltpu.VMEM((_CH,), jnp.int32),
            pltpu.VMEM((_CBLK, _HB, _W), jnp.float32),
            pltpu.VMEM((_CBLK, _HB, _W), jnp.float32),
            pltpu.SemaphoreType.DMA,
            pltpu.SemaphoreType.DMA,
            pltpu.SemaphoreType.DMA,
            pltpu.SemaphoreType.DMA,
            pltpu.SemaphoreType.DMA,
        ],
    )
    def k(tt_hbm, idx_hbm, out_hbm, lut_v, idx0, idx1, st0, st1,
          sem_lut, sem_i0, sem_i1, sem_o0, sem_o1):
        wid = lax.axis_index("s") * _NC + lax.axis_index("c")
        cblk = wid // (_NW // _NCB)
        bpair = wid % (_NW // _NCB)
        idx_bufs = (idx0, idx1)
        stages = (st0, st1)
        sem_is = (sem_i0, sem_i1)
        sem_os = (sem_o0, sem_o1)

        def bh(t):
            return bpair * _BPW + t // _NCHUNK, (t % _NCHUNK) * _HB

        def idx_off(t):
            b, h0 = bh(t)
            return b * _HW + h0 * _W

        lut_copy = pltpu.async_copy(
            tt_hbm.at[pl.ds(cblk * (_CBLK * _VP), _CBLK * _VP)], lut_v,
            sem_lut)
        pltpu.async_copy(idx_hbm.at[pl.ds(idx_off(0), _CH)], idx0, sem_i0)
        pltpu.async_copy(idx_hbm.at[pl.ds(idx_off(1), _CH)], idx1, sem_i1)
        lut_copy.wait()

        def chunk_body(t, _):
            for s in (0, 1):
                te = t + s
                idx_v, stage_v = idx_bufs[s], stages[s]
                pltpu.make_async_copy(
                    idx_hbm.at[pl.ds(0, _CH)], idx_v, sem_is[s]).wait()
                @pl.when(te >= 2)
                def _drain_out():
                    pltpu.make_async_copy(
                        stage_v,
                        out_hbm.at[0, pl.ds(0, _CBLK), pl.ds(0, _HB), :],
                        sem_os[s]).wait()

                @plsc.parallel_loop(0, _HB * _WG, unroll=4)
                def _pos_body(i):
                    r = i // _WG
                    w0 = (i % _WG) * 16
                    iv = idx_v[pl.ds(i * 16, 16)]
                    for c in range(_CBLK):
                        stage_v[c, r, pl.ds(w0, 16)] = (
                            plsc.load_gather(lut_v, [iv + c * _VP]))

                b, h0 = bh(te)
                pltpu.async_copy(
                    stage_v,
                    out_hbm.at[b, pl.ds(cblk * _CBLK, _CBLK), pl.ds(h0, _HB), :],
                    sem_os[s])
                @pl.when(te + 2 < _T)
                def _prefetch():
                    pltpu.async_copy(
                        idx_hbm.at[pl.ds(idx_off(te + 2), _CH)], idx_v,
                        sem_is[s])
            return 0

        lax.fori_loop(0, _T // 2, lambda u, c: chunk_body(u * 2, c), 0)
        for s in (0, 1):
            pltpu.make_async_copy(
                stages[s], out_hbm.at[0, pl.ds(0, _CBLK), pl.ds(0, _HB), :],
                sem_os[s]).wait()

    return k(table_t, idx)


def kernel(x, table):
    idx = x.reshape(_B * _HW)
    tpad = jnp.zeros((_VP, _C), jnp.float32).at[:_V].set(table)
    table_t = _transpose_table(tpad).reshape(_C * _VP)
    return _sc_gather(table_t, idx)


# speedup vs baseline: 1.9613x; 1.2407x over previous
import functools

import jax
import jax.numpy as jnp
from jax import lax
from jax.experimental import pallas as pl
from jax.experimental.pallas import tpu as pltpu
from jax.experimental.pallas import tpu_sc as plsc

_B, _C, _H, _W = 8, 128, 224, 224
_HW = _H * _W
_N = _B * _HW
_NC, _NS = 2, 16
_NW = _NC * _NS
_RPW = _N // _NW
_GS = 112
_K = 2 * _GS
_T = _RPW // _K


def _sc_gather(table, idx):
    mesh = plsc.VectorSubcoreMesh(
        core_axis_name="c", subcore_axis_name="s",
        num_cores=_NC, num_subcores=_NS)

    @functools.partial(
        pl.kernel,
        out_type=jax.ShapeDtypeStruct((_N, _C), jnp.float32),
        mesh=mesh,
        compiler_params=pltpu.CompilerParams(needs_layout_passes=False),
        scratch_types=[
            pltpu.VMEM((_K,), jnp.int32),
            pltpu.VMEM((_K,), jnp.int32),
            pltpu.VMEM((_K, _C), jnp.float32),
            pltpu.VMEM((_K, _C), jnp.float32),
            pltpu.SemaphoreType.DMA,
            pltpu.SemaphoreType.DMA,
            pltpu.SemaphoreType.DMA,
            pltpu.SemaphoreType.DMA,
            pltpu.SemaphoreType.DMA,
            pltpu.SemaphoreType.DMA,
        ],
    )
    def k(tab_hbm, idx_hbm, out_hbm, idx0, idx1, rows0, rows1,
          sem_i0, sem_i1, sem_g0, sem_g1, sem_o0, sem_o1):
        wid = lax.axis_index("s") * _NC + lax.axis_index("c")
        base = wid * _RPW
        idx_bufs = (idx0, idx1)
        row_bufs = (rows0, rows1)
        sem_is = (sem_i0, sem_i1)
        sem_gs = (sem_g0, sem_g1)
        sem_os = (sem_o0, sem_o1)

        pltpu.async_copy(idx_hbm.at[pl.ds(base, _K)], idx0, sem_i0)
        pltpu.async_copy(idx_hbm.at[pl.ds(base + _K, _K)], idx1, sem_i1)

        def chunk_body(t, _):
            for s in (0, 1):
                te = t + s
                idx_v, rows_v = idx_bufs[s], row_bufs[s]
                r0 = base + te * _K
                pltpu.make_async_copy(
                    idx_hbm.at[pl.ds(0, _K)], idx_v, sem_is[s]).wait()
                @pl.when(te >= 2)
                def _drain_out():
                    pltpu.make_async_copy(
                        rows_v, out_hbm.at[pl.ds(0, _K)], sem_os[s]).wait()

                for g in range(_K // _GS):
                    pltpu.async_copy(
                        tab_hbm.at[idx_v.at[pl.ds(g * _GS, _GS)]],
                        rows_v.at[pl.ds(g * _GS, _GS)], sem_gs[s])
                for g in range(_K // _GS):
                    pltpu.make_async_copy(
                        tab_hbm.at[idx_v.at[pl.ds(0, _GS)]],
                        rows_v.at[pl.ds(0, _GS)], sem_gs[s]).wait()
                pltpu.async_copy(rows_v, out_hbm.at[pl.ds(r0, _K)], sem_os[s])
                @pl.when(te + 2 < _T)
                def _prefetch():
                    pltpu.async_copy(
                        idx_hbm.at[pl.ds(base + (te + 2) * _K, _K)], idx_v,
                        sem_is[s])
            return 0

        lax.fori_loop(0, _T // 2, lambda u, c: chunk_body(u * 2, c), 0)
        for s in (0, 1):
            pltpu.make_async_copy(
                row_bufs[s], out_hbm.at[pl.ds(0, _K)], sem_os[s]).wait()

    return k(table, idx)


def kernel(x, table):
    idx = x.reshape(_N)
    emb = _sc_gather(table, idx)
    emb = emb.reshape(_B, _H, _W, _C)
    return jnp.transpose(emb, (0, 3, 1, 2))
